# baseline (speedup 1.0000x reference)
import functools

import jax
import jax.numpy as jnp
from jax import lax
from jax.experimental import pallas as pl
from jax.experimental.pallas import tpu as pltpu
from jax.experimental.pallas import tpu_sc as plsc

B = 16384
D = 64
_L = 16

_info = plsc.get_sparse_core_info()
_NC, _NS = _info.num_cores, _info.num_subcores
_NW = _NC * _NS
_RPW = B // _NW
_GROUPS = _RPW // _L


def _make_kernel():
    mesh = plsc.VectorSubcoreMesh(core_axis_name="c", subcore_axis_name="s")

    @functools.partial(
        pl.kernel,
        mesh=mesh,
        out_type=[
            jax.ShapeDtypeStruct((B,), jnp.float32),
            jax.ShapeDtypeStruct((B * D,), jnp.float32),
            jax.ShapeDtypeStruct((B * D,), jnp.float32),
        ],
        scratch_types=[
            pltpu.VMEM((_RPW * D,), jnp.float32),
            pltpu.VMEM((_RPW * D,), jnp.float32),
            pltpu.VMEM((_RPW,), jnp.float32),
            pltpu.SemaphoreType.DMA,
            pltpu.SemaphoreType.DMA,
        ],
        compiler_params=pltpu.CompilerParams(needs_layout_passes=False),
    )
    def dot_kernel(gu_hbm, gi_hbm, xui_hbm, gout_u, gout_i, u_v, i_v, o_v,
                   sem_in, sem_out):
        wid = lax.axis_index("s") * _NC + lax.axis_index("c")
        base = wid * _RPW
        cu = pltpu.async_copy(gu_hbm.at[pl.ds(base * D, _RPW * D)], u_v, sem_in)
        ci = pltpu.async_copy(gi_hbm.at[pl.ds(base * D, _RPW * D)], i_v, sem_in)
        cu.wait()
        ci.wait()
        ou = pltpu.async_copy(u_v, gout_u.at[pl.ds(base * D, _RPW * D)], sem_out)
        oi = pltpu.async_copy(i_v, gout_i.at[pl.ds(base * D, _RPW * D)], sem_out)

        lanes = lax.iota(jnp.int32, _L)

        def group_body(g, carry):
            rows = (g * _L + lanes) * D

            def col_body(j, acc):
                idx = rows + j
                a = plsc.load_gather(u_v, [idx])
                b = plsc.load_gather(i_v, [idx])
                return acc + a * b

            acc = lax.fori_loop(0, D, col_body, jnp.zeros((_L,), jnp.float32))
            o_v[pl.ds(g * _L, _L)] = acc
            return carry

        lax.fori_loop(0, _GROUPS, group_body, 0)
        pltpu.sync_copy(o_v, xui_hbm.at[pl.ds(base, _RPW)])
        ou.wait()
        oi.wait()

    return dot_kernel


_dot = _make_kernel()


def kernel(gu, gi):
    xui, gamma_u, gamma_i = _dot(gu.reshape(B * D), gi.reshape(B * D))
    return (xui, gamma_u.reshape(B, D), gamma_i.reshape(B, D))
